# baseline (speedup 1.0000x reference)
import functools

import jax
import jax.numpy as jnp
from jax import lax
from jax.experimental import pallas as pl
from jax.experimental.pallas import tpu as pltpu
from jax.experimental.pallas import tpu_sc as plsc

VOCAB = 1000000
EMBED = 64
B = 4096
L = 200
EPS = 1e-5

BL = B * L
SR = BL // 2
NSLICE = 4
SRS = SR // NSLICE
NC, NS = 2, 16
NW = NC * NS
SR_PER_W = SRS // NW
CS = 320
NCHUNK = SR_PER_W // CS
GS = 64
NSTR = 2 * (CS // GS)
NBUF = 2

TCBLK = 6400
TCB = 2 * TCBLK // L
SLICE_BLKS = SRS // TCBLK


def _sc_body(ids_hbm, tok_hbm, out_hbm, idx_v, rows_v, sem0, sem1):
    wid = lax.axis_index("s") * NC + lax.axis_index("c")
    sems = (sem0, sem1)

    def start_gather(c, b):
        idx_row = 2 * (wid * (SR_PER_W // GS) + c * (CS // GS))
        pltpu.sync_copy(ids_hbm.at[pl.ds(idx_row, NSTR)],
                        idx_v.at[pl.ds(b * NSTR, NSTR)])
        for j in range(NSTR):
            gg, h = j // 2, j % 2
            pltpu.async_copy(
                tok_hbm.at[idx_v.at[b * NSTR + j]],
                rows_v.at[pl.ds((b * 2 + h) * CS + gg * GS, GS)],
                sems[b])

    def drain(b):
        pltpu.make_async_copy(
            tok_hbm.at[pl.ds(0, 2 * CS)],
            rows_v.at[pl.ds(b * 2 * CS, 2 * CS)], sems[b]).wait()

    def writeback(c, b):
        out_base = wid * SR_PER_W + c * CS
        for h in range(2):
            pltpu.sync_copy(
                rows_v.at[pl.ds((b * 2 + h) * CS, CS)],
                out_hbm.at[pl.ds(out_base, CS), pl.ds(64 * h, 64)])

    start_gather(0, 0)

    def pair_body(i, carry):
        c2 = i * NBUF
        for b in range(NBUF):
            c = c2 + b

            @pl.when(c < NCHUNK - 1)
            def _():
                start_gather(c + 1, (b + 1) % NBUF)
            drain(b)
            writeback(c, b)
        return carry
    lax.fori_loop(0, NCHUNK // NBUF, pair_body, 0)


def _ln_body(y_ref, pos_ref, m_ref, g_ref, b_ref, _, o_ref):
    x = y_ref[...] + jnp.tile(pos_ref[...], (TCBLK // L, 1))
    m = m_ref[...]
    mu = jnp.dot(x, m, preferred_element_type=jnp.float32)
    var = jnp.dot(x * x, m, preferred_element_type=jnp.float32) - mu * mu
    o = ((x - mu) * lax.rsqrt(var + EPS) * g_ref[0:1, :]
         + b_ref[0:1, :])
    o2 = jnp.concatenate([o[:, :EMBED], o[:, EMBED:]], axis=0)
    o_ref[...] = o2.reshape(TCB, L, EMBED)


def _make_gather():
    mesh = plsc.VectorSubcoreMesh(core_axis_name="c", subcore_axis_name="s")
    return functools.partial(
        pl.kernel,
        mesh=mesh,
        out_type=jax.ShapeDtypeStruct((SRS, 128), jnp.float32),
        compiler_params=pltpu.CompilerParams(
            needs_layout_passes=False, use_tc_tiling_on_sc=False),
        scratch_types=[
            pltpu.VMEM((NBUF * NSTR, GS), jnp.int32),
            pltpu.VMEM((NBUF * 2 * CS, EMBED), jnp.float32),
            pltpu.SemaphoreType.DMA,
            pltpu.SemaphoreType.DMA,
        ],
    )(_sc_body)


def _make_ln(i):
    return pl.pallas_call(
        _ln_body,
        grid=(SLICE_BLKS,),
        in_specs=[
            pl.BlockSpec((TCBLK, 128), lambda j: (j, 0)),
            pl.BlockSpec((L, 128), lambda j: (0, 0)),
            pl.BlockSpec((128, 128), lambda j: (0, 0)),
            pl.BlockSpec((8, 128), lambda j: (0, 0)),
            pl.BlockSpec((8, 128), lambda j: (0, 0)),
            pl.BlockSpec(memory_space=pl.ANY),
        ],
        out_specs=pl.BlockSpec((TCB, L, EMBED),
                               lambda j, i=i: (i * SLICE_BLKS + j, 0, 0)),
        out_shape=jax.ShapeDtypeStruct((B, L, EMBED), jnp.float32),
        input_output_aliases={5: 0},
    )


@jax.jit
def _call(idsD, token_table, pos2, mmat, g2, b2):
    gather = _make_gather()
    out = jnp.zeros((B, L, EMBED), jnp.float32)
    rows_per_slice = 2 * SRS // GS
    for i in range(NSLICE):
        y = gather(lax.slice_in_dim(idsD, i * rows_per_slice,
                                    (i + 1) * rows_per_slice), token_table)
        out = _make_ln(i)(y, pos2, mmat, g2, b2, out)
    return out


def kernel(input_ids, token_table, pos_table, ln_gamma, ln_beta):
    ids = input_ids.reshape(-1).astype(jnp.int32)
    f2 = ids.reshape(SR // TCBLK, 2, TCBLK)
    idsD = (f2.transpose(0, 2, 1)
            .reshape(SR // GS, GS, 2)
            .transpose(0, 2, 1)
            .reshape(2 * (SR // GS), GS))
    pos2 = jnp.concatenate([pos_table[:L]] * 2, axis=1)
    eye2 = jnp.eye(2, dtype=jnp.float32)
    mmat = jnp.kron(eye2, jnp.full((EMBED, EMBED), 1.0 / EMBED, jnp.float32))
    g2 = jnp.tile(jnp.concatenate([ln_gamma, ln_gamma])[None, :], (8, 1))
    b2 = jnp.tile(jnp.concatenate([ln_beta, ln_beta])[None, :], (8, 1))
    return _call(idsD, token_table, pos2, mmat, g2, b2)


# speedup vs baseline: 1.1040x; 1.1040x over previous
import functools

import jax
import jax.numpy as jnp
from jax import lax
from jax.experimental import pallas as pl
from jax.experimental.pallas import tpu as pltpu
from jax.experimental.pallas import tpu_sc as plsc

VOCAB = 1000000
EMBED = 64
B = 4096
L = 200
EPS = 1e-5

BL = B * L
SR = BL // 2
NC, NS = 2, 16
NW = NC * NS
SR_PER_W = SR // NW
CS = 320
NCHUNK = SR_PER_W // CS
GS = 64
NSTR = 2 * (CS // GS)
NBUF = 2

TCBLK = 12800
TCGRID = SR // TCBLK


def _sc_body(ids_hbm, tok_hbm, out_hbm, idx_v, rows_v, sem0, sem1):
    wid = lax.axis_index("s") * NC + lax.axis_index("c")
    sems = (sem0, sem1)

    def start_gather(c, b):
        idx_row = 2 * (wid * (SR_PER_W // GS) + c * (CS // GS))
        pltpu.sync_copy(ids_hbm.at[pl.ds(idx_row, NSTR)],
                        idx_v.at[pl.ds(b * NSTR, NSTR)])
        for j in range(NSTR):
            gg, h = j // 2, j % 2
            pltpu.async_copy(
                tok_hbm.at[idx_v.at[b * NSTR + j]],
                rows_v.at[pl.ds((b * 2 + h) * CS + gg * GS, GS)],
                sems[b])

    def drain(b):
        pltpu.make_async_copy(
            tok_hbm.at[pl.ds(0, 2 * CS)],
            rows_v.at[pl.ds(b * 2 * CS, 2 * CS)], sems[b]).wait()

    def writeback(c, b):
        out_base = wid * SR_PER_W + c * CS
        for h in range(2):
            pltpu.sync_copy(
                rows_v.at[pl.ds((b * 2 + h) * CS, CS)],
                out_hbm.at[pl.ds(out_base, CS), pl.ds(64 * h, 64)])

    start_gather(0, 0)

    def pair_body(i, carry):
        c2 = i * NBUF
        for b in range(NBUF):
            c = c2 + b

            @pl.when(c < NCHUNK - 1)
            def _():
                start_gather(c + 1, (b + 1) % NBUF)
            drain(b)
            writeback(c, b)
        return carry
    lax.fori_loop(0, NCHUNK // NBUF, pair_body, 0)


def _ln_body(y_ref, pos_ref, m_ref, g_ref, b_ref, o_ref):
    x = y_ref[...] + jnp.tile(pos_ref[...], (TCBLK // L, 1))
    m = m_ref[...]
    mu = jnp.dot(x, m, preferred_element_type=jnp.float32)
    var = jnp.dot(x * x, m, preferred_element_type=jnp.float32) - mu * mu
    o = ((x - mu) * lax.rsqrt(var + EPS) * g_ref[0:1, :]
         + b_ref[0:1, :])
    o2 = jnp.concatenate([o[:, :EMBED], o[:, EMBED:]], axis=0)
    o_ref[...] = o2.reshape(2 * TCBLK // L, L, EMBED)


@jax.jit
def _call(ids, token_table, pos2, mmat, g2, b2):
    mesh = plsc.VectorSubcoreMesh(core_axis_name="c", subcore_axis_name="s")
    gather = functools.partial(
        pl.kernel,
        mesh=mesh,
        out_type=jax.ShapeDtypeStruct((SR, 128), jnp.float32),
        compiler_params=pltpu.CompilerParams(
            needs_layout_passes=False, use_tc_tiling_on_sc=False),
        scratch_types=[
            pltpu.VMEM((NBUF * NSTR, GS), jnp.int32),
            pltpu.VMEM((NBUF * 2 * CS, EMBED), jnp.float32),
            pltpu.SemaphoreType.DMA,
            pltpu.SemaphoreType.DMA,
        ],
    )(_sc_body)
    y = gather(ids, token_table)

    out = pl.pallas_call(
        _ln_body,
        grid=(TCGRID,),
        in_specs=[
            pl.BlockSpec((TCBLK, 128), lambda i: (i, 0)),
            pl.BlockSpec((L, 128), lambda i: (0, 0)),
            pl.BlockSpec((128, 128), lambda i: (0, 0)),
            pl.BlockSpec((8, 128), lambda i: (0, 0)),
            pl.BlockSpec((8, 128), lambda i: (0, 0)),
        ],
        out_specs=pl.BlockSpec((2 * TCBLK // L, L, EMBED), lambda i: (i, 0, 0)),
        out_shape=jax.ShapeDtypeStruct((B, L, EMBED), jnp.float32),
    )(y, pos2, mmat, g2, b2)
    return out


def kernel(input_ids, token_table, pos_table, ln_gamma, ln_beta):
    ids = input_ids.reshape(-1).astype(jnp.int32)
    f2 = ids.reshape(SR // TCBLK, 2, TCBLK)
    idsD = (f2.transpose(0, 2, 1)
            .reshape(SR // GS, GS, 2)
            .transpose(0, 2, 1)
            .reshape(2 * (SR // GS), GS))
    pos2 = jnp.concatenate([pos_table[:L]] * 2, axis=1)
    eye2 = jnp.eye(2, dtype=jnp.float32)
    mmat = jnp.kron(eye2, jnp.full((EMBED, EMBED), 1.0 / EMBED, jnp.float32))
    g2 = jnp.tile(jnp.concatenate([ln_gamma, ln_gamma])[None, :], (8, 1))
    b2 = jnp.tile(jnp.concatenate([ln_beta, ln_beta])[None, :], (8, 1))
    return _call(idsD, token_table, pos2, mmat, g2, b2)
